# initial kernel scaffold (unmeasured)
import jax
import jax.numpy as jnp
from jax import lax
from jax.experimental import pallas as pl
from jax.experimental.pallas import tpu as pltpu

T = 4096
TOK_HALF = T // 2


def kernel(ids, E):
    v_shard, d = E.shape
    my_x = lax.axis_index("x")
    my_y = lax.axis_index("y")

    ids_half = lax.dynamic_slice(ids, (my_x * TOK_HALF,), (TOK_HALF,))
    local = ids_half - my_y * v_shard
    valid = (local >= 0) & (local < v_shard)
    safe = jnp.where(valid, local, 0)
    partial = jnp.where(valid[:, None], E[safe], jnp.float32(0.0))

    def body(partial_ref, out_ref, recv_ref, sems):
        x = lax.axis_index("x")
        y = lax.axis_index("y")

        barrier = pltpu.get_barrier_semaphore()
        pl.semaphore_signal(barrier, inc=1, device_id=(x, 1 - y),
                            device_id_type=pl.DeviceIdType.MESH)
        pl.semaphore_signal(barrier, inc=1, device_id=(1 - x, y),
                            device_id_type=pl.DeviceIdType.MESH)
        pl.semaphore_wait(barrier, 2)

        rdma1 = pltpu.make_async_remote_copy(
            src_ref=partial_ref,
            dst_ref=recv_ref,
            send_sem=sems.at[0],
            recv_sem=sems.at[1],
            device_id=(x, 1 - y),
            device_id_type=pl.DeviceIdType.MESH,
        )
        rdma1.start()
        rdma1.wait()

        off = x * TOK_HALF
        out_ref[pl.ds(off, TOK_HALF), :] = partial_ref[...] + recv_ref[...]

        rdma2 = pltpu.make_async_remote_copy(
            src_ref=out_ref.at[pl.ds(off, TOK_HALF)],
            dst_ref=out_ref.at[pl.ds(off, TOK_HALF)],
            send_sem=sems.at[2],
            recv_sem=sems.at[3],
            device_id=(1 - x, y),
            device_id_type=pl.DeviceIdType.MESH,
        )
        rdma2.start()
        rdma2.wait()

    return pl.pallas_call(
        body,
        out_shape=jax.ShapeDtypeStruct((T, d), jnp.float32),
        in_specs=[pl.BlockSpec(memory_space=pltpu.VMEM)],
        out_specs=pl.BlockSpec(memory_space=pltpu.VMEM),
        scratch_shapes=[
            pltpu.VMEM((TOK_HALF, d), jnp.float32),
            pltpu.SemaphoreType.DMA((4,)),
        ],
        compiler_params=pltpu.CompilerParams(collective_id=0),
    )(partial)


# baseline (device time: 1330378 ns/iter reference)
import jax
import jax.numpy as jnp
from jax import lax
from jax.experimental import pallas as pl
from jax.experimental.pallas import tpu as pltpu

T = 4096
TOK_HALF = T // 2


def kernel(ids, E):
    v_shard, d = E.shape
    my_x = lax.axis_index("x")
    my_y = lax.axis_index("y")

    ids_half = lax.dynamic_slice(ids, (my_x * TOK_HALF,), (TOK_HALF,))
    local = ids_half - my_y * v_shard
    valid = (local >= 0) & (local < v_shard)
    safe = jnp.where(valid, local, 0)
    partial = jnp.where(valid[:, None], E[safe], jnp.float32(0.0))

    def body(partial_ref, out_ref, sems):
        x = lax.axis_index("x")
        y = lax.axis_index("y")
        off = x * TOK_HALF

        barrier = pltpu.get_barrier_semaphore()
        pl.semaphore_signal(barrier, inc=1, device_id=(x, 1 - y),
                            device_id_type=pl.DeviceIdType.MESH)
        pl.semaphore_signal(barrier, inc=1, device_id=(1 - x, y),
                            device_id_type=pl.DeviceIdType.MESH)
        pl.semaphore_wait(barrier, 2)

        rdma1 = pltpu.make_async_remote_copy(
            src_ref=partial_ref,
            dst_ref=out_ref.at[pl.ds(off, TOK_HALF)],
            send_sem=sems.at[0],
            recv_sem=sems.at[1],
            device_id=(x, 1 - y),
            device_id_type=pl.DeviceIdType.MESH,
        )
        rdma1.start()
        rdma1.wait()

        out_ref[pl.ds(off, TOK_HALF), :] = (
            out_ref[pl.ds(off, TOK_HALF), :] + partial_ref[...]
        )

        rdma2 = pltpu.make_async_remote_copy(
            src_ref=out_ref.at[pl.ds(off, TOK_HALF)],
            dst_ref=out_ref.at[pl.ds(off, TOK_HALF)],
            send_sem=sems.at[2],
            recv_sem=sems.at[3],
            device_id=(1 - x, y),
            device_id_type=pl.DeviceIdType.MESH,
        )
        rdma2.start()
        rdma2.wait()

    return pl.pallas_call(
        body,
        out_shape=jax.ShapeDtypeStruct((T, d), jnp.float32),
        in_specs=[pl.BlockSpec(memory_space=pltpu.VMEM)],
        out_specs=pl.BlockSpec(memory_space=pltpu.VMEM),
        scratch_shapes=[
            pltpu.SemaphoreType.DMA((4,)),
        ],
        compiler_params=pltpu.CompilerParams(
            collective_id=0,
            vmem_limit_bytes=100 * 1024 * 1024,
        ),
    )(partial)


# device time: 514938 ns/iter; 2.5836x vs baseline; 2.5836x over previous
import jax
import jax.numpy as jnp
from jax import lax
from jax.experimental import pallas as pl
from jax.experimental.pallas import tpu as pltpu

T = 4096
TOK_HALF = T // 2
NBUF = 16


def kernel(ids, E):
    v_shard, d = E.shape
    my_x = lax.axis_index("x")
    my_y = lax.axis_index("y")

    ids_half = lax.dynamic_slice(ids, (my_x * TOK_HALF,), (TOK_HALF,))
    local = ids_half - my_y * v_shard
    valid = (local >= 0) & (local < v_shard)
    safe = jnp.where(valid, local, 0).astype(jnp.int32)
    mask = valid.astype(jnp.float32)[:, None]

    def body(safe_ref, mask_ref, E_ref, out_ref, gbuf_ref, gsems, sems):
        x = lax.axis_index("x")
        y = lax.axis_index("y")
        off = x * TOK_HALF

        barrier = pltpu.get_barrier_semaphore()
        pl.semaphore_signal(barrier, inc=1, device_id=(x, 1 - y),
                            device_id_type=pl.DeviceIdType.MESH)
        pl.semaphore_signal(barrier, inc=1, device_id=(1 - x, y),
                            device_id_type=pl.DeviceIdType.MESH)

        def issue(i, slot):
            pltpu.make_async_copy(
                E_ref.at[pl.ds(safe_ref[i], 1), :],
                gbuf_ref.at[pl.ds(i, 1), :],
                gsems.at[slot],
            ).start()

        def wait_slot(slot):
            pltpu.make_async_copy(
                E_ref.at[pl.ds(0, 1), :],
                gbuf_ref.at[pl.ds(0, 1), :],
                gsems.at[slot],
            ).wait()

        def prologue(i, c):
            issue(i, i)
            return c
        lax.fori_loop(0, NBUF, prologue, 0)

        def steady(i, c):
            slot = lax.rem(i, NBUF)
            wait_slot(slot)
            issue(i, slot)
            return c
        lax.fori_loop(NBUF, TOK_HALF, steady, 0)

        def drain(i, c):
            wait_slot(lax.rem(i, NBUF))
            return c
        lax.fori_loop(TOK_HALF, TOK_HALF + NBUF, drain, 0)

        pl.semaphore_wait(barrier, 2)

        rdma1 = pltpu.make_async_remote_copy(
            src_ref=gbuf_ref,
            dst_ref=out_ref.at[pl.ds(off, TOK_HALF)],
            send_sem=sems.at[0],
            recv_sem=sems.at[1],
            device_id=(x, 1 - y),
            device_id_type=pl.DeviceIdType.MESH,
        )
        rdma1.start()
        rdma1.wait()

        out_ref[pl.ds(off, TOK_HALF), :] = jnp.where(
            mask_ref[...] != 0.0,
            gbuf_ref[...],
            out_ref[pl.ds(off, TOK_HALF), :],
        )

        rdma2 = pltpu.make_async_remote_copy(
            src_ref=out_ref.at[pl.ds(off, TOK_HALF)],
            dst_ref=out_ref.at[pl.ds(off, TOK_HALF)],
            send_sem=sems.at[2],
            recv_sem=sems.at[3],
            device_id=(1 - x, y),
            device_id_type=pl.DeviceIdType.MESH,
        )
        rdma2.start()
        rdma2.wait()

    return pl.pallas_call(
        body,
        out_shape=jax.ShapeDtypeStruct((T, d), jnp.float32),
        in_specs=[
            pl.BlockSpec(memory_space=pltpu.SMEM),
            pl.BlockSpec(memory_space=pltpu.VMEM),
            pl.BlockSpec(memory_space=pl.ANY),
        ],
        out_specs=pl.BlockSpec(memory_space=pltpu.VMEM),
        scratch_shapes=[
            pltpu.VMEM((TOK_HALF, d), jnp.float32),
            pltpu.SemaphoreType.DMA((NBUF,)),
            pltpu.SemaphoreType.DMA((4,)),
        ],
        compiler_params=pltpu.CompilerParams(
            collective_id=0,
            vmem_limit_bytes=100 * 1024 * 1024,
        ),
    )(safe, mask, E)


# device time: 334767 ns/iter; 3.9740x vs baseline; 1.5382x over previous
import jax
import jax.numpy as jnp
from jax import lax
from jax.experimental import pallas as pl
from jax.experimental.pallas import tpu as pltpu

T = 4096
TOK_HALF = T // 2
NBUF = 16
C = 8
CHUNK = TOK_HALF // C


def kernel(ids, E):
    v_shard, d = E.shape
    my_x = lax.axis_index("x")
    my_y = lax.axis_index("y")

    ids_half = lax.dynamic_slice(ids, (my_x * TOK_HALF,), (TOK_HALF,))
    local = ids_half - my_y * v_shard
    valid = (local >= 0) & (local < v_shard)
    safe = jnp.where(valid, local, 0).astype(jnp.int32)
    mask = valid.astype(jnp.float32)[:, None]

    def body(safe_ref, mask_ref, E_ref, out_ref, gbuf_ref, gsems,
             s1send, s1recv, s2send, s2recv):
        x = lax.axis_index("x")
        y = lax.axis_index("y")
        off = x * TOK_HALF

        barrier = pltpu.get_barrier_semaphore()
        pl.semaphore_signal(barrier, inc=1, device_id=(x, 1 - y),
                            device_id_type=pl.DeviceIdType.MESH)
        pl.semaphore_signal(barrier, inc=1, device_id=(1 - x, y),
                            device_id_type=pl.DeviceIdType.MESH)

        def issue(i, slot):
            pltpu.make_async_copy(
                E_ref.at[pl.ds(safe_ref[i], 1), :],
                gbuf_ref.at[pl.ds(i, 1), :],
                gsems.at[slot],
            ).start()

        def wait_slot(slot):
            pltpu.make_async_copy(
                E_ref.at[pl.ds(0, 1), :],
                gbuf_ref.at[pl.ds(0, 1), :],
                gsems.at[slot],
            ).wait()

        def gather_chunk(c):
            base = c * CHUNK

            def prologue(k, t):
                issue(base + k, k)
                return t
            lax.fori_loop(0, NBUF, prologue, 0)

            def steady(k, t):
                slot = lax.rem(k, NBUF)
                wait_slot(slot)
                issue(base + k, slot)
                return t
            lax.fori_loop(NBUF, CHUNK, steady, 0)

            def drain(k, t):
                wait_slot(lax.rem(k, NBUF))
                return t
            lax.fori_loop(CHUNK, CHUNK + NBUF, drain, 0)

        def rdma1(c):
            return pltpu.make_async_remote_copy(
                src_ref=gbuf_ref.at[pl.ds(c * CHUNK, CHUNK)],
                dst_ref=out_ref.at[pl.ds(off + c * CHUNK, CHUNK)],
                send_sem=s1send.at[c],
                recv_sem=s1recv.at[c],
                device_id=(x, 1 - y),
                device_id_type=pl.DeviceIdType.MESH,
            )

        def rdma2(c):
            return pltpu.make_async_remote_copy(
                src_ref=out_ref.at[pl.ds(off + c * CHUNK, CHUNK)],
                dst_ref=out_ref.at[pl.ds(off + c * CHUNK, CHUNK)],
                send_sem=s2send.at[c],
                recv_sem=s2recv.at[c],
                device_id=(1 - x, y),
                device_id_type=pl.DeviceIdType.MESH,
            )

        gather_chunk(0)
        pl.semaphore_wait(barrier, 2)
        rdma1(0).start()
        for c in range(1, C):
            gather_chunk(c)
            rdma1(c).start()

        for c in range(C):
            rdma1(c).wait_recv()
            sl = pl.ds(off + c * CHUNK, CHUNK)
            out_ref[sl, :] = jnp.where(
                mask_ref[c * CHUNK:(c + 1) * CHUNK, :] != 0.0,
                gbuf_ref[c * CHUNK:(c + 1) * CHUNK, :],
                out_ref[sl, :],
            )
            rdma2(c).start()

        for c in range(C):
            rdma2(c).wait_recv()
        for c in range(C):
            rdma1(c).wait_send()
            rdma2(c).wait_send()

    return pl.pallas_call(
        body,
        out_shape=jax.ShapeDtypeStruct((T, d), jnp.float32),
        in_specs=[
            pl.BlockSpec(memory_space=pltpu.SMEM),
            pl.BlockSpec(memory_space=pltpu.VMEM),
            pl.BlockSpec(memory_space=pl.ANY),
        ],
        out_specs=pl.BlockSpec(memory_space=pltpu.VMEM),
        scratch_shapes=[
            pltpu.VMEM((TOK_HALF, d), jnp.float32),
            pltpu.SemaphoreType.DMA((NBUF,)),
            pltpu.SemaphoreType.DMA((C,)),
            pltpu.SemaphoreType.DMA((C,)),
            pltpu.SemaphoreType.DMA((C,)),
            pltpu.SemaphoreType.DMA((C,)),
        ],
        compiler_params=pltpu.CompilerParams(
            collective_id=0,
            vmem_limit_bytes=100 * 1024 * 1024,
        ),
    )(safe, mask, E)


# device time: 278419 ns/iter; 4.7783x vs baseline; 1.2024x over previous
import jax
import jax.numpy as jnp
from jax import lax
from jax.experimental import pallas as pl
from jax.experimental.pallas import tpu as pltpu

T = 4096
TOK_HALF = T // 2
NBUF = 64
C = 8
CHUNK = TOK_HALF // C


def kernel(ids, E):
    v_shard, d = E.shape
    my_x = lax.axis_index("x")
    my_y = lax.axis_index("y")

    ids_half = lax.dynamic_slice(ids, (my_x * TOK_HALF,), (TOK_HALF,))
    local = ids_half - my_y * v_shard
    valid = (local >= 0) & (local < v_shard)
    safe = jnp.where(valid, local, 0).astype(jnp.int32)
    mask = valid.astype(jnp.float32)[:, None]

    def body(safe_ref, mask_ref, E_ref, out_ref, gbuf_ref, gsems,
             s1send, s1recv, s2send, s2recv):
        x = lax.axis_index("x")
        y = lax.axis_index("y")
        off = x * TOK_HALF

        barrier = pltpu.get_barrier_semaphore()
        pl.semaphore_signal(barrier, inc=1, device_id=(x, 1 - y),
                            device_id_type=pl.DeviceIdType.MESH)
        pl.semaphore_signal(barrier, inc=1, device_id=(1 - x, y),
                            device_id_type=pl.DeviceIdType.MESH)

        def issue(i, slot):
            pltpu.make_async_copy(
                E_ref.at[pl.ds(safe_ref[i], 1), :],
                gbuf_ref.at[pl.ds(i, 1), :],
                gsems.at[slot],
            ).start()

        def wait_slot(slot):
            pltpu.make_async_copy(
                E_ref.at[pl.ds(0, 1), :],
                gbuf_ref.at[pl.ds(0, 1), :],
                gsems.at[slot],
            ).wait()

        def gather_chunk(c):
            base = c * CHUNK

            def prologue(k, t):
                issue(base + k, k)
                return t
            lax.fori_loop(0, NBUF, prologue, 0)

            def steady(k, t):
                slot = lax.rem(k, NBUF)
                wait_slot(slot)
                issue(base + k, slot)
                return t
            lax.fori_loop(NBUF, CHUNK, steady, 0)

            def drain(k, t):
                wait_slot(lax.rem(k, NBUF))
                return t
            lax.fori_loop(CHUNK, CHUNK + NBUF, drain, 0)

        def rdma1(c):
            return pltpu.make_async_remote_copy(
                src_ref=gbuf_ref.at[pl.ds(c * CHUNK, CHUNK)],
                dst_ref=out_ref.at[pl.ds(off + c * CHUNK, CHUNK)],
                send_sem=s1send.at[c],
                recv_sem=s1recv.at[c],
                device_id=(x, 1 - y),
                device_id_type=pl.DeviceIdType.MESH,
            )

        def rdma2(c):
            return pltpu.make_async_remote_copy(
                src_ref=out_ref.at[pl.ds(off + c * CHUNK, CHUNK)],
                dst_ref=out_ref.at[pl.ds(off + c * CHUNK, CHUNK)],
                send_sem=s2send.at[c],
                recv_sem=s2recv.at[c],
                device_id=(1 - x, y),
                device_id_type=pl.DeviceIdType.MESH,
            )

        gather_chunk(0)
        pl.semaphore_wait(barrier, 2)
        rdma1(0).start()
        for c in range(1, C):
            gather_chunk(c)
            rdma1(c).start()

        for c in range(C):
            rdma1(c).wait_recv()
            sl = pl.ds(off + c * CHUNK, CHUNK)
            out_ref[sl, :] = jnp.where(
                mask_ref[c * CHUNK:(c + 1) * CHUNK, :] != 0.0,
                gbuf_ref[c * CHUNK:(c + 1) * CHUNK, :],
                out_ref[sl, :],
            )
            rdma2(c).start()

        for c in range(C):
            rdma2(c).wait_recv()
        for c in range(C):
            rdma1(c).wait_send()
            rdma2(c).wait_send()

    return pl.pallas_call(
        body,
        out_shape=jax.ShapeDtypeStruct((T, d), jnp.float32),
        in_specs=[
            pl.BlockSpec(memory_space=pltpu.SMEM),
            pl.BlockSpec(memory_space=pltpu.VMEM),
            pl.BlockSpec(memory_space=pl.ANY),
        ],
        out_specs=pl.BlockSpec(memory_space=pltpu.VMEM),
        scratch_shapes=[
            pltpu.VMEM((TOK_HALF, d), jnp.float32),
            pltpu.SemaphoreType.DMA((NBUF,)),
            pltpu.SemaphoreType.DMA((C,)),
            pltpu.SemaphoreType.DMA((C,)),
            pltpu.SemaphoreType.DMA((C,)),
            pltpu.SemaphoreType.DMA((C,)),
        ],
        compiler_params=pltpu.CompilerParams(
            collective_id=0,
            vmem_limit_bytes=100 * 1024 * 1024,
        ),
    )(safe, mask, E)


# device time: 269141 ns/iter; 4.9431x vs baseline; 1.0345x over previous
import jax
import jax.numpy as jnp
from jax import lax
from jax.experimental import pallas as pl
from jax.experimental.pallas import tpu as pltpu

T = 4096
TOK_HALF = T // 2
NBUF = 64
C = 8
CHUNK = TOK_HALF // C


def kernel(ids, E):
    v_shard, d = E.shape
    my_x = lax.axis_index("x")
    my_y = lax.axis_index("y")

    ids_half = lax.dynamic_slice(ids, (my_x * TOK_HALF,), (TOK_HALF,))
    local = ids_half - my_y * v_shard
    valid = (local >= 0) & (local < v_shard)
    safe = jnp.where(valid, local, 0).astype(jnp.int32)
    mask = valid.astype(jnp.float32)[:, None]

    chunk_id = jnp.arange(TOK_HALF, dtype=jnp.int32) // CHUNK
    key = jnp.where(valid, chunk_id, C)
    ow_pos = jnp.argsort(key, stable=True).astype(jnp.int32)
    ow_row = safe[ow_pos]
    cnt = jnp.sum(valid.reshape(C, CHUNK), axis=1, dtype=jnp.int32)
    starts = jnp.concatenate(
        [jnp.zeros((1,), jnp.int32), jnp.cumsum(cnt).astype(jnp.int32)]
    )

    def body(ow_pos_ref, ow_row_ref, starts_ref, mask_ref, E_ref, out_ref,
             gbuf_ref, gsems, s1send, s1recv, s2send, s2recv):
        x = lax.axis_index("x")
        y = lax.axis_index("y")
        off = x * TOK_HALF

        barrier = pltpu.get_barrier_semaphore()
        pl.semaphore_signal(barrier, inc=1, device_id=(x, 1 - y),
                            device_id_type=pl.DeviceIdType.MESH)
        pl.semaphore_signal(barrier, inc=1, device_id=(1 - x, y),
                            device_id_type=pl.DeviceIdType.MESH)

        def issue(j, slot):
            pltpu.make_async_copy(
                E_ref.at[pl.ds(ow_row_ref[j], 1), :],
                gbuf_ref.at[pl.ds(ow_pos_ref[j], 1), :],
                gsems.at[slot],
            ).start()

        def wait_slot(slot):
            pltpu.make_async_copy(
                E_ref.at[pl.ds(0, 1), :],
                gbuf_ref.at[pl.ds(0, 1), :],
                gsems.at[slot],
            ).wait()

        def gather_chunk(c):
            s0 = starts_ref[c]
            n = starts_ref[c + 1] - s0

            def step(k, t):
                slot = lax.rem(k, NBUF)

                @pl.when(k >= NBUF)
                def _():
                    wait_slot(slot)
                issue(s0 + k, slot)
                return t
            lax.fori_loop(0, n, step, 0)

            def drain(k, t):
                wait_slot(lax.rem(k, NBUF))
                return t
            lax.fori_loop(jnp.maximum(n - NBUF, 0), n, drain, 0)

        def rdma1(c):
            return pltpu.make_async_remote_copy(
                src_ref=gbuf_ref.at[pl.ds(c * CHUNK, CHUNK)],
                dst_ref=out_ref.at[pl.ds(off + c * CHUNK, CHUNK)],
                send_sem=s1send.at[c],
                recv_sem=s1recv.at[c],
                device_id=(x, 1 - y),
                device_id_type=pl.DeviceIdType.MESH,
            )

        def rdma2(c):
            return pltpu.make_async_remote_copy(
                src_ref=out_ref.at[pl.ds(off + c * CHUNK, CHUNK)],
                dst_ref=out_ref.at[pl.ds(off + c * CHUNK, CHUNK)],
                send_sem=s2send.at[c],
                recv_sem=s2recv.at[c],
                device_id=(1 - x, y),
                device_id_type=pl.DeviceIdType.MESH,
            )

        gather_chunk(0)
        pl.semaphore_wait(barrier, 2)
        rdma1(0).start()
        for c in range(1, C):
            gather_chunk(c)
            rdma1(c).start()

        for c in range(C):
            rdma1(c).wait_recv()
            sl = pl.ds(off + c * CHUNK, CHUNK)
            out_ref[sl, :] = jnp.where(
                mask_ref[c * CHUNK:(c + 1) * CHUNK, :] != 0.0,
                gbuf_ref[c * CHUNK:(c + 1) * CHUNK, :],
                out_ref[sl, :],
            )
            rdma2(c).start()

        for c in range(C):
            rdma2(c).wait_recv()
        for c in range(C):
            rdma1(c).wait_send()
            rdma2(c).wait_send()

    return pl.pallas_call(
        body,
        out_shape=jax.ShapeDtypeStruct((T, d), jnp.float32),
        in_specs=[
            pl.BlockSpec(memory_space=pltpu.SMEM),
            pl.BlockSpec(memory_space=pltpu.SMEM),
            pl.BlockSpec(memory_space=pltpu.SMEM),
            pl.BlockSpec(memory_space=pltpu.VMEM),
            pl.BlockSpec(memory_space=pl.ANY),
        ],
        out_specs=pl.BlockSpec(memory_space=pltpu.VMEM),
        scratch_shapes=[
            pltpu.VMEM((TOK_HALF, d), jnp.float32),
            pltpu.SemaphoreType.DMA((NBUF,)),
            pltpu.SemaphoreType.DMA((C,)),
            pltpu.SemaphoreType.DMA((C,)),
            pltpu.SemaphoreType.DMA((C,)),
            pltpu.SemaphoreType.DMA((C,)),
        ],
        compiler_params=pltpu.CompilerParams(
            collective_id=0,
            vmem_limit_bytes=100 * 1024 * 1024,
        ),
    )(ow_pos, ow_row, starts, mask, E)
